# Initial kernel scaffold; baseline (speedup 1.0000x reference)
import jax
import jax.numpy as jnp
from jax.experimental import pallas as pl


def kernel(x, edge_index, batch, W1, b1, W2, b2, Wl, bl):
    raise NotImplementedError("write your pallas kernel here")


# speedup vs baseline: 19.0489x; 19.0489x over previous
import functools

import jax
import jax.numpy as jnp
from jax import lax
from jax.experimental import pallas as pl
from jax.experimental.pallas import tpu as pltpu
from jax.experimental.pallas import tpu_sc as plsc

N = 10000
E = 320000
D = 128
H = 16
G = 64
NPAD = 10240
CHUNK = 128
NCHUNKS = E // CHUNK
NW = 32
ROWS_PER_SUB = NPAD // 16
BLK = 256
GRID = NPAD // BLK

_mesh = plsc.VectorSubcoreMesh(core_axis_name="c", subcore_axis_name="s")
_sc_params = pltpu.CompilerParams(use_tc_tiling_on_sc=False)


@functools.partial(
    pl.kernel,
    out_type=jax.ShapeDtypeStruct((2, NPAD), jnp.float32),
    mesh=_mesh,
    compiler_params=_sc_params,
    scratch_types=[
        pltpu.VMEM_SHARED((NPAD,), jnp.float32),
        pltpu.VMEM((CHUNK,), jnp.int32),
        pltpu.VMEM((CHUNK,), jnp.float32),
    ],
)
def _deg(dst_hbm, zeros1_hbm, degp_hbm, deg_sh, idx_v, ones_v):
    c = lax.axis_index("c")
    s = lax.axis_index("s")
    wid = s * 2 + c
    for k in range(CHUNK // 16):
        ones_v[pl.ds(k * 16, 16)] = jnp.full((16,), 1.0, jnp.float32)
    pltpu.sync_copy(zeros1_hbm.at[pl.ds(s * ROWS_PER_SUB, ROWS_PER_SUB)],
                    deg_sh.at[pl.ds(s * ROWS_PER_SUB, ROWS_PER_SUB)])
    plsc.subcore_barrier()

    def body(i, carry):
        ch = wid + i * NW

        @pl.when(ch < NCHUNKS)
        def _():
            pltpu.sync_copy(dst_hbm.at[pl.ds(ch * CHUNK, CHUNK)], idx_v)
            pltpu.sync_copy(ones_v, deg_sh.at[idx_v], add=True)
        return carry

    lax.fori_loop(0, (NCHUNKS + NW - 1) // NW, body, 0)
    plsc.subcore_barrier()
    pltpu.sync_copy(deg_sh.at[pl.ds(s * ROWS_PER_SUB, ROWS_PER_SUB)],
                    degp_hbm.at[c].at[pl.ds(s * ROWS_PER_SUB, ROWS_PER_SUB)])


@functools.partial(
    pl.kernel,
    out_type=jax.ShapeDtypeStruct((2, NPAD, H), jnp.float32),
    mesh=_mesh,
    compiler_params=_sc_params,
    scratch_types=[
        pltpu.VMEM_SHARED((NPAD, H), jnp.float32),
        pltpu.VMEM((CHUNK,), jnp.int32),
        pltpu.VMEM((CHUNK,), jnp.int32),
        pltpu.VMEM((CHUNK, H), jnp.float32),
        pltpu.SemaphoreType.DMA,
    ],
)
def _msg(src_hbm, dst_hbm, g_hbm, zeros2_hbm, sp_hbm,
         acc_sh, sidx_v, didx_v, rows_v, sem):
    c = lax.axis_index("c")
    s = lax.axis_index("s")
    wid = s * 2 + c
    pltpu.sync_copy(zeros2_hbm.at[pl.ds(s * ROWS_PER_SUB, ROWS_PER_SUB)],
                    acc_sh.at[pl.ds(s * ROWS_PER_SUB, ROWS_PER_SUB)])
    plsc.subcore_barrier()

    def body(i, carry):
        ch = wid + i * NW

        @pl.when(ch < NCHUNKS)
        def _():
            e0 = ch * CHUNK
            pltpu.sync_copy(src_hbm.at[pl.ds(e0, CHUNK)], sidx_v)
            pltpu.sync_copy(dst_hbm.at[pl.ds(e0, CHUNK)], didx_v)
            pltpu.async_copy(g_hbm.at[sidx_v], rows_v, sem).wait()
            pltpu.sync_copy(rows_v, acc_sh.at[didx_v], add=True)
        return carry

    lax.fori_loop(0, (NCHUNKS + NW - 1) // NW, body, 0)
    plsc.subcore_barrier()
    pltpu.sync_copy(acc_sh.at[pl.ds(s * ROWS_PER_SUB, ROWS_PER_SUB)],
                    sp_hbm.at[c].at[pl.ds(s * ROWS_PER_SUB, ROWS_PER_SUB)])


def _mm1_body(x_ref, w1_ref, d0_ref, d1_ref, g_ref, dinv_ref):
    deg = d0_ref[...] + d1_ref[...] + 1.0
    dinv = lax.rsqrt(jnp.maximum(deg, 1.0))
    h = jnp.dot(x_ref[...], w1_ref[...], preferred_element_type=jnp.float32)
    g_ref[...] = h * dinv
    dinv_ref[...] = dinv


_mm1 = pl.pallas_call(
    _mm1_body,
    grid=(GRID,),
    in_specs=[
        pl.BlockSpec((BLK, D), lambda i: (i, 0)),
        pl.BlockSpec((D, H), lambda i: (0, 0)),
        pl.BlockSpec((BLK, 1), lambda i: (i, 0)),
        pl.BlockSpec((BLK, 1), lambda i: (i, 0)),
    ],
    out_specs=[
        pl.BlockSpec((BLK, H), lambda i: (i, 0)),
        pl.BlockSpec((BLK, 1), lambda i: (i, 0)),
    ],
    out_shape=[
        jax.ShapeDtypeStruct((NPAD, H), jnp.float32),
        jax.ShapeDtypeStruct((NPAD, 1), jnp.float32),
    ],
)


def _comb1_body(s0_ref, s1_ref, g_ref, dinv_ref, w2_ref, b1_ref, g2_ref):
    h1 = jnp.maximum(
        (s0_ref[...] + s1_ref[...] + g_ref[...]) * dinv_ref[...] + b1_ref[...],
        0.0)
    g2_ref[...] = jnp.dot(h1, w2_ref[...],
                          preferred_element_type=jnp.float32) * dinv_ref[...]


_comb1 = pl.pallas_call(
    _comb1_body,
    grid=(GRID,),
    in_specs=[
        pl.BlockSpec((BLK, H), lambda i: (i, 0)),
        pl.BlockSpec((BLK, H), lambda i: (i, 0)),
        pl.BlockSpec((BLK, H), lambda i: (i, 0)),
        pl.BlockSpec((BLK, 1), lambda i: (i, 0)),
        pl.BlockSpec((H, H), lambda i: (0, 0)),
        pl.BlockSpec((1, H), lambda i: (0, 0)),
    ],
    out_specs=pl.BlockSpec((BLK, H), lambda i: (i, 0)),
    out_shape=jax.ShapeDtypeStruct((NPAD, H), jnp.float32),
)


def _comb2_body(q0_ref, q1_ref, g2_ref, dinv_ref, b2_ref, batch_ref,
                wl_ref, bl_ref, out_ref, pool_acc, cnt_acc):
    i = pl.program_id(0)

    @pl.when(i == 0)
    def _():
        pool_acc[...] = jnp.zeros_like(pool_acc)
        cnt_acc[...] = jnp.zeros_like(cnt_acc)

    h2 = jnp.maximum(
        (q0_ref[...] + q1_ref[...] + g2_ref[...]) * dinv_ref[...] + b2_ref[...],
        0.0)
    iota = lax.broadcasted_iota(jnp.int32, (BLK, G), 1)
    onehot = (batch_ref[...] == iota).astype(jnp.float32)
    pool_acc[...] += lax.dot_general(
        onehot, h2, (((0,), (0,)), ((), ())),
        preferred_element_type=jnp.float32)
    cnt_acc[...] += lax.dot_general(
        onehot, jnp.ones((BLK, 1), jnp.float32), (((0,), (0,)), ((), ())),
        preferred_element_type=jnp.float32)

    @pl.when(i == GRID - 1)
    def _():
        pooled = pool_acc[...] / jnp.maximum(cnt_acc[...], 1.0)
        z = jnp.dot(pooled, wl_ref[...],
                    preferred_element_type=jnp.float32) + bl_ref[...]
        out_ref[...] = jax.nn.sigmoid(z)


_comb2 = pl.pallas_call(
    _comb2_body,
    grid=(GRID,),
    in_specs=[
        pl.BlockSpec((BLK, H), lambda i: (i, 0)),
        pl.BlockSpec((BLK, H), lambda i: (i, 0)),
        pl.BlockSpec((BLK, H), lambda i: (i, 0)),
        pl.BlockSpec((BLK, 1), lambda i: (i, 0)),
        pl.BlockSpec((1, H), lambda i: (0, 0)),
        pl.BlockSpec((BLK, 1), lambda i: (i, 0)),
        pl.BlockSpec((H, 1), lambda i: (0, 0)),
        pl.BlockSpec((1, 1), lambda i: (0, 0)),
    ],
    out_specs=pl.BlockSpec((G, 1), lambda i: (0, 0)),
    out_shape=jax.ShapeDtypeStruct((G, 1), jnp.float32),
    scratch_shapes=[
        pltpu.VMEM((G, H), jnp.float32),
        pltpu.VMEM((G, 1), jnp.float32),
    ],
)


def kernel(x, edge_index, batch, W1, b1, W2, b2, Wl, bl):
    src = edge_index[0]
    dst = edge_index[1]
    xpad = jnp.pad(x, ((0, NPAD - N), (0, 0)))
    batchp = jnp.pad(batch, (0, NPAD - N),
                     constant_values=G + 1).reshape(NPAD, 1)
    zeros1 = jnp.zeros((NPAD,), jnp.float32)
    zeros2 = jnp.zeros((NPAD, H), jnp.float32)

    degp = _deg(dst, zeros1)
    d0 = degp[0].reshape(NPAD, 1)
    d1 = degp[1].reshape(NPAD, 1)
    g1, dinv = _mm1(xpad, W1, d0, d1)

    sp1 = _msg(src, dst, g1, zeros2)
    g2 = _comb1(sp1[0], sp1[1], g1, dinv, W2, b1.reshape(1, H))

    sp2 = _msg(src, dst, g2, zeros2)
    out2d = _comb2(sp2[0], sp2[1], g2, dinv, b2.reshape(1, H), batchp,
                   Wl, bl.reshape(1, 1))
    return out2d[:, 0]


# speedup vs baseline: 42.0824x; 2.2092x over previous
import functools

import jax
import jax.numpy as jnp
from jax import lax
from jax.experimental import pallas as pl
from jax.experimental.pallas import tpu as pltpu
from jax.experimental.pallas import tpu_sc as plsc

N = 10000
E = 320000
D = 128
H = 16
G = 64
NPAD = 10240
CHUNK = 128
NCHUNKS = E // CHUNK
NW = 32
ROWS_PER_SUB = NPAD // 16
K78 = NCHUNKS // NW
TAIL = NCHUNKS - K78 * NW
NBUF = 4
BLK = 256
GRID = NPAD // BLK

_mesh = plsc.VectorSubcoreMesh(core_axis_name="c", subcore_axis_name="s")
_sc_params = pltpu.CompilerParams(use_tc_tiling_on_sc=False)


@functools.partial(
    pl.kernel,
    out_type=jax.ShapeDtypeStruct((2, NPAD), jnp.float32),
    mesh=_mesh,
    compiler_params=_sc_params,
    scratch_types=[
        pltpu.VMEM_SHARED((NPAD,), jnp.float32),
        pltpu.VMEM((K78 + 1, CHUNK), jnp.int32),
        pltpu.VMEM((CHUNK,), jnp.float32),
        pltpu.SemaphoreType.DMA,
        pltpu.SemaphoreType.DMA,
        pltpu.SemaphoreType.DMA,
        pltpu.SemaphoreType.DMA,
    ],
)
def _deg(dst2_hbm, zeros1_hbm, degp_hbm, deg_sh, didx, ones_v,
         sm0, sm1, sm2, sm3):
    ssem = (sm0, sm1, sm2, sm3)
    c = lax.axis_index("c")
    s = lax.axis_index("s")
    wid = s * 2 + c
    for k in range(CHUNK // 16):
        ones_v[pl.ds(k * 16, 16)] = jnp.full((16,), 1.0, jnp.float32)
    pltpu.sync_copy(zeros1_hbm.at[pl.ds(s * ROWS_PER_SUB, ROWS_PER_SUB)],
                    deg_sh.at[pl.ds(s * ROWS_PER_SUB, ROWS_PER_SUB)])
    pltpu.sync_copy(dst2_hbm.at[pl.ds(wid * K78, K78)],
                    didx.at[pl.ds(0, K78)])

    @pl.when(wid < TAIL)
    def _():
        pltpu.sync_copy(dst2_hbm.at[K78 * NW + wid], didx.at[K78])

    kmax = jnp.where(wid < TAIL, K78 + 1, K78)
    plsc.subcore_barrier()

    def grp_body(gidx, carry):
        for b in range(NBUF):
            j = gidx * NBUF + b

            @pl.when(j < kmax)
            def _(b=b, j=j):
                @pl.when(j >= NBUF)
                def _():
                    pltpu.make_async_copy(
                        ones_v, deg_sh.at[didx.at[0]], ssem[b]).wait()
                pltpu.async_copy(ones_v, deg_sh.at[didx.at[j]], ssem[b],
                                 add=True)
        return carry

    lax.fori_loop(0, (K78 + 1 + NBUF - 1) // NBUF, grp_body, 0)
    for b in range(NBUF):
        pltpu.make_async_copy(ones_v, deg_sh.at[didx.at[0]], ssem[b]).wait()
    plsc.subcore_barrier()
    pltpu.sync_copy(deg_sh.at[pl.ds(s * ROWS_PER_SUB, ROWS_PER_SUB)],
                    degp_hbm.at[c].at[pl.ds(s * ROWS_PER_SUB, ROWS_PER_SUB)])


@functools.partial(
    pl.kernel,
    out_type=jax.ShapeDtypeStruct((2, NPAD, H), jnp.float32),
    mesh=_mesh,
    compiler_params=_sc_params,
    scratch_types=[
        pltpu.VMEM_SHARED((NPAD, H), jnp.float32),
        pltpu.VMEM((K78 + 1, CHUNK), jnp.int32),
        pltpu.VMEM((K78 + 1, CHUNK), jnp.int32),
        pltpu.VMEM((CHUNK, H), jnp.float32),
        pltpu.VMEM((CHUNK, H), jnp.float32),
        pltpu.VMEM((CHUNK, H), jnp.float32),
        pltpu.VMEM((CHUNK, H), jnp.float32),
        pltpu.SemaphoreType.DMA,
        pltpu.SemaphoreType.DMA,
        pltpu.SemaphoreType.DMA,
        pltpu.SemaphoreType.DMA,
        pltpu.SemaphoreType.DMA,
        pltpu.SemaphoreType.DMA,
        pltpu.SemaphoreType.DMA,
        pltpu.SemaphoreType.DMA,
    ],
)
def _msg(src2_hbm, dst2_hbm, g_hbm, zeros2_hbm, sp_hbm,
         acc_sh, sidx, didx, r0, r1, r2, r3,
         gm0, gm1, gm2, gm3, sm0, sm1, sm2, sm3):
    rows = (r0, r1, r2, r3)
    gsem = (gm0, gm1, gm2, gm3)
    ssem = (sm0, sm1, sm2, sm3)
    c = lax.axis_index("c")
    s = lax.axis_index("s")
    wid = s * 2 + c
    pltpu.sync_copy(zeros2_hbm.at[pl.ds(s * ROWS_PER_SUB, ROWS_PER_SUB)],
                    acc_sh.at[pl.ds(s * ROWS_PER_SUB, ROWS_PER_SUB)])
    pltpu.sync_copy(src2_hbm.at[pl.ds(wid * K78, K78)],
                    sidx.at[pl.ds(0, K78)])
    pltpu.sync_copy(dst2_hbm.at[pl.ds(wid * K78, K78)],
                    didx.at[pl.ds(0, K78)])

    @pl.when(wid < TAIL)
    def _():
        pltpu.sync_copy(src2_hbm.at[K78 * NW + wid], sidx.at[K78])
        pltpu.sync_copy(dst2_hbm.at[K78 * NW + wid], didx.at[K78])

    kmax = jnp.where(wid < TAIL, K78 + 1, K78)
    plsc.subcore_barrier()

    def issue_gather(j, b):
        pltpu.async_copy(g_hbm.at[sidx.at[j]], rows[b], gsem[b])

    def wait_gather(b):
        pltpu.make_async_copy(g_hbm.at[sidx.at[0]], rows[b], gsem[b]).wait()

    def issue_scatter(j, b):
        pltpu.async_copy(rows[b], acc_sh.at[didx.at[j]], ssem[b], add=True)

    def wait_scatter(b):
        pltpu.make_async_copy(rows[b], acc_sh.at[didx.at[0]], ssem[b]).wait()

    for b in range(NBUF):
        issue_gather(jnp.int32(b), b)

    def grp_body(gidx, carry):
        for b in range(NBUF):
            j = gidx * NBUF + b

            @pl.when(j < kmax)
            def _(b=b, j=j):
                wait_gather(b)
                issue_scatter(j, b)
                jn = j + NBUF

                @pl.when(jn < kmax)
                def _(b=b, jn=jn):
                    wait_scatter(b)
                    issue_gather(jn, b)
        return carry

    lax.fori_loop(0, (K78 + 1 + NBUF - 1) // NBUF, grp_body, 0)
    for b in range(NBUF):
        wait_scatter(b)
    plsc.subcore_barrier()
    pltpu.sync_copy(acc_sh.at[pl.ds(s * ROWS_PER_SUB, ROWS_PER_SUB)],
                    sp_hbm.at[c].at[pl.ds(s * ROWS_PER_SUB, ROWS_PER_SUB)])


def _mm1_body(x_ref, w1_ref, d0_ref, d1_ref, g_ref, dinv_ref):
    deg = d0_ref[...] + d1_ref[...] + 1.0
    dinv = lax.rsqrt(jnp.maximum(deg, 1.0))
    h = jnp.dot(x_ref[...], w1_ref[...], preferred_element_type=jnp.float32)
    g_ref[...] = h * dinv
    dinv_ref[...] = dinv


_mm1 = pl.pallas_call(
    _mm1_body,
    grid=(GRID,),
    in_specs=[
        pl.BlockSpec((BLK, D), lambda i: (i, 0)),
        pl.BlockSpec((D, H), lambda i: (0, 0)),
        pl.BlockSpec((BLK, 1), lambda i: (i, 0)),
        pl.BlockSpec((BLK, 1), lambda i: (i, 0)),
    ],
    out_specs=[
        pl.BlockSpec((BLK, H), lambda i: (i, 0)),
        pl.BlockSpec((BLK, 1), lambda i: (i, 0)),
    ],
    out_shape=[
        jax.ShapeDtypeStruct((NPAD, H), jnp.float32),
        jax.ShapeDtypeStruct((NPAD, 1), jnp.float32),
    ],
)


def _comb1_body(s0_ref, s1_ref, g_ref, dinv_ref, w2_ref, b1_ref, g2_ref):
    h1 = jnp.maximum(
        (s0_ref[...] + s1_ref[...] + g_ref[...]) * dinv_ref[...] + b1_ref[...],
        0.0)
    g2_ref[...] = jnp.dot(h1, w2_ref[...],
                          preferred_element_type=jnp.float32) * dinv_ref[...]


_comb1 = pl.pallas_call(
    _comb1_body,
    grid=(GRID,),
    in_specs=[
        pl.BlockSpec((BLK, H), lambda i: (i, 0)),
        pl.BlockSpec((BLK, H), lambda i: (i, 0)),
        pl.BlockSpec((BLK, H), lambda i: (i, 0)),
        pl.BlockSpec((BLK, 1), lambda i: (i, 0)),
        pl.BlockSpec((H, H), lambda i: (0, 0)),
        pl.BlockSpec((1, H), lambda i: (0, 0)),
    ],
    out_specs=pl.BlockSpec((BLK, H), lambda i: (i, 0)),
    out_shape=jax.ShapeDtypeStruct((NPAD, H), jnp.float32),
)


def _comb2_body(q0_ref, q1_ref, g2_ref, dinv_ref, b2_ref, batch_ref,
                wl_ref, bl_ref, out_ref, pool_acc, cnt_acc):
    i = pl.program_id(0)

    @pl.when(i == 0)
    def _():
        pool_acc[...] = jnp.zeros_like(pool_acc)
        cnt_acc[...] = jnp.zeros_like(cnt_acc)

    h2 = jnp.maximum(
        (q0_ref[...] + q1_ref[...] + g2_ref[...]) * dinv_ref[...] + b2_ref[...],
        0.0)
    iota = lax.broadcasted_iota(jnp.int32, (BLK, G), 1)
    onehot = (batch_ref[...] == iota).astype(jnp.float32)
    pool_acc[...] += lax.dot_general(
        onehot, h2, (((0,), (0,)), ((), ())),
        preferred_element_type=jnp.float32)
    cnt_acc[...] += lax.dot_general(
        onehot, jnp.ones((BLK, 1), jnp.float32), (((0,), (0,)), ((), ())),
        preferred_element_type=jnp.float32)

    @pl.when(i == GRID - 1)
    def _():
        pooled = pool_acc[...] / jnp.maximum(cnt_acc[...], 1.0)
        z = jnp.dot(pooled, wl_ref[...],
                    preferred_element_type=jnp.float32) + bl_ref[...]
        out_ref[...] = jax.nn.sigmoid(z)


_comb2 = pl.pallas_call(
    _comb2_body,
    grid=(GRID,),
    in_specs=[
        pl.BlockSpec((BLK, H), lambda i: (i, 0)),
        pl.BlockSpec((BLK, H), lambda i: (i, 0)),
        pl.BlockSpec((BLK, H), lambda i: (i, 0)),
        pl.BlockSpec((BLK, 1), lambda i: (i, 0)),
        pl.BlockSpec((1, H), lambda i: (0, 0)),
        pl.BlockSpec((BLK, 1), lambda i: (i, 0)),
        pl.BlockSpec((H, 1), lambda i: (0, 0)),
        pl.BlockSpec((1, 1), lambda i: (0, 0)),
    ],
    out_specs=pl.BlockSpec((G, 1), lambda i: (0, 0)),
    out_shape=jax.ShapeDtypeStruct((G, 1), jnp.float32),
    scratch_shapes=[
        pltpu.VMEM((G, H), jnp.float32),
        pltpu.VMEM((G, 1), jnp.float32),
    ],
)


def kernel(x, edge_index, batch, W1, b1, W2, b2, Wl, bl):
    src2 = edge_index[0].reshape(NCHUNKS, CHUNK)
    dst2 = edge_index[1].reshape(NCHUNKS, CHUNK)
    xpad = jnp.pad(x, ((0, NPAD - N), (0, 0)))
    batchp = jnp.pad(batch, (0, NPAD - N),
                     constant_values=G + 1).reshape(NPAD, 1)
    zeros1 = jnp.zeros((NPAD,), jnp.float32)
    zeros2 = jnp.zeros((NPAD, H), jnp.float32)

    degp = _deg(dst2, zeros1)
    d0 = degp[0].reshape(NPAD, 1)
    d1 = degp[1].reshape(NPAD, 1)
    g1, dinv = _mm1(xpad, W1, d0, d1)

    sp1 = _msg(src2, dst2, g1, zeros2)
    g2 = _comb1(sp1[0], sp1[1], g1, dinv, W2, b1.reshape(1, H))

    sp2 = _msg(src2, dst2, g2, zeros2)
    out2d = _comb2(sp2[0], sp2[1], g2, dinv, b2.reshape(1, H), batchp,
                   Wl, bl.reshape(1, 1))
    return out2d[:, 0]
